# Initial kernel scaffold; baseline (speedup 1.0000x reference)
import jax
import jax.numpy as jnp
from jax.experimental import pallas as pl


def kernel(x, adj, poly_item, filter_param):
    raise NotImplementedError("write your pallas kernel here")


# speedup vs baseline: 1.6651x; 1.6651x over previous
import math

import jax
import jax.numpy as jnp
from jax.experimental import pallas as pl
from jax.experimental.pallas import tpu as pltpu

_BM = 256


def _pass_body(s_ref, m_ref, v_ref, u_ref, o_ref):
    a = s_ref[0]
    c = s_ref[1]
    o_ref[...] = a * jnp.dot(
        m_ref[...], v_ref[...], preferred_element_type=jnp.float32
    ) + c * u_ref[...]


def _apply(m, v, u, a, c):
    n = m.shape[0]
    d = v.shape[1]
    s = jnp.stack([a, c]).astype(jnp.float32)
    return pl.pallas_call(
        _pass_body,
        grid=(n // _BM,),
        in_specs=[
            pl.BlockSpec(memory_space=pltpu.SMEM),
            pl.BlockSpec((_BM, n), lambda i: (i, 0)),
            pl.BlockSpec((n, d), lambda i: (0, 0)),
            pl.BlockSpec((_BM, d), lambda i: (i, 0)),
        ],
        out_specs=pl.BlockSpec((_BM, d), lambda i: (i, 0)),
        out_shape=jax.ShapeDtypeStruct((n, d), jnp.float32),
    )(s, m, v, u)


def kernel(x, adj, poly_item, filter_param):
    k = filter_param.shape[0] - 1
    fp = jax.nn.relu(filter_param)[:, 0]
    coefs = [fp[i] * (math.comb(k, i) / (2.0 ** k)) for i in range(k + 1)]
    one = jnp.float32(1.0)
    zero = jnp.float32(0.0)

    us = [x]
    for _ in range(k):
        us.append(_apply(adj, us[-1], x, one, zero))

    z = _apply(poly_item, x, us[1], coefs[k], coefs[k - 1])
    for j in range(1, k):
        z = _apply(poly_item, z, us[j + 1], one, coefs[k - 1 - j])
    return z


# speedup vs baseline: 1.7886x; 1.0742x over previous
import math

import jax
import jax.numpy as jnp
from jax.experimental import pallas as pl
from jax.experimental.pallas import tpu as pltpu

_BM = 256


def _cast_pass_body(s_ref, m_ref, v_ref, u_ref, o_ref, mbf_ref):
    a = s_ref[0]
    c = s_ref[1]
    mb = m_ref[...]
    mbf_ref[...] = mb.astype(jnp.bfloat16)
    o_ref[...] = a * jnp.dot(
        mb, v_ref[...], preferred_element_type=jnp.float32
    ) + c * u_ref[...]


def _bf_pass_body(s_ref, m_ref, v_ref, u_ref, o_ref):
    a = s_ref[0]
    c = s_ref[1]
    o_ref[...] = a * jnp.dot(
        m_ref[...], v_ref[...].astype(jnp.bfloat16),
        preferred_element_type=jnp.float32,
    ) + c * u_ref[...]


def _apply_cast(m, v, u, a, c):
    n = m.shape[0]
    d = v.shape[1]
    s = jnp.stack([a, c]).astype(jnp.float32)
    return pl.pallas_call(
        _cast_pass_body,
        grid=(n // _BM,),
        in_specs=[
            pl.BlockSpec(memory_space=pltpu.SMEM),
            pl.BlockSpec((_BM, n), lambda i: (i, 0)),
            pl.BlockSpec((n, d), lambda i: (0, 0)),
            pl.BlockSpec((_BM, d), lambda i: (i, 0)),
        ],
        out_specs=[
            pl.BlockSpec((_BM, d), lambda i: (i, 0)),
            pl.BlockSpec((_BM, n), lambda i: (i, 0)),
        ],
        out_shape=[
            jax.ShapeDtypeStruct((n, d), jnp.float32),
            jax.ShapeDtypeStruct((n, n), jnp.bfloat16),
        ],
    )(s, m, v, u)


def _apply_bf(m_bf, v, u, a, c):
    n = m_bf.shape[0]
    d = v.shape[1]
    s = jnp.stack([a, c]).astype(jnp.float32)
    return pl.pallas_call(
        _bf_pass_body,
        grid=(n // _BM,),
        in_specs=[
            pl.BlockSpec(memory_space=pltpu.SMEM),
            pl.BlockSpec((_BM, n), lambda i: (i, 0)),
            pl.BlockSpec((n, d), lambda i: (0, 0)),
            pl.BlockSpec((_BM, d), lambda i: (i, 0)),
        ],
        out_specs=pl.BlockSpec((_BM, d), lambda i: (i, 0)),
        out_shape=jax.ShapeDtypeStruct((n, d), jnp.float32),
    )(s, m_bf, v, u)


def kernel(x, adj, poly_item, filter_param):
    k = filter_param.shape[0] - 1
    fp = jax.nn.relu(filter_param)[:, 0]
    coefs = [fp[i] * (math.comb(k, i) / (2.0 ** k)) for i in range(k + 1)]
    one = jnp.float32(1.0)
    zero = jnp.float32(0.0)

    u1, adj_bf = _apply_cast(adj, x, x, one, zero)
    us = [x, u1]
    for _ in range(1, k):
        us.append(_apply_bf(adj_bf, us[-1], x, one, zero))

    z, poly_bf = _apply_cast(poly_item, x, us[1], coefs[k], coefs[k - 1])
    for j in range(1, k):
        z = _apply_bf(poly_bf, z, us[j + 1], one, coefs[k - 1 - j])
    return z


# speedup vs baseline: 1.8530x; 1.0360x over previous
import functools
import math

import jax
import jax.numpy as jnp
from jax.experimental import pallas as pl
from jax.experimental.pallas import tpu as pltpu

_BM = 256


def _cast_pass_body(s_ref, m_ref, v_ref, u_ref, o_ref, mbf_ref):
    a = s_ref[0]
    c = s_ref[1]
    mb = m_ref[...]
    mbf_ref[...] = mb.astype(jnp.bfloat16)
    o_ref[...] = a * jnp.dot(
        mb, v_ref[...], preferred_element_type=jnp.float32
    ) + c * u_ref[...]


def _apply_cast(m, v, u, a, c):
    n = m.shape[0]
    d = v.shape[1]
    s = jnp.stack([a, c]).astype(jnp.float32)
    return pl.pallas_call(
        _cast_pass_body,
        grid=(n // _BM,),
        in_specs=[
            pl.BlockSpec(memory_space=pltpu.SMEM),
            pl.BlockSpec((_BM, n), lambda i: (i, 0)),
            pl.BlockSpec((n, d), lambda i: (0, 0)),
            pl.BlockSpec((_BM, d), lambda i: (i, 0)),
        ],
        out_specs=[
            pl.BlockSpec((_BM, d), lambda i: (i, 0)),
            pl.BlockSpec((_BM, n), lambda i: (i, 0)),
        ],
        out_shape=[
            jax.ShapeDtypeStruct((n, d), jnp.float32),
            jax.ShapeDtypeStruct((n, n), jnp.bfloat16),
        ],
    )(s, m, v, u)


def _fused_body(k, s_ref, abf_ref, pbf_ref, u1_ref, z1_ref, o_ref,
                ub0, ub1, zb0, zb1):
    p = pl.program_id(0)
    r = pl.program_id(1)
    rows = pl.ds(r * _BM, _BM)
    npass = 2 * (k - 1)
    ubufs = (ub0, ub1)
    zbufs = (zb0, zb1)

    for pp in range(npass):
        if pp % 2 == 0:
            i = pp // 2

            @pl.when(p == pp)
            def _(i=i):
                src = u1_ref[...] if i == 0 else ubufs[(i + 1) % 2][...]
                ubufs[i % 2][rows, :] = jnp.dot(
                    abf_ref[...], src.astype(jnp.bfloat16),
                    preferred_element_type=jnp.float32,
                )
        else:
            j = (pp + 1) // 2

            @pl.when(p == pp)
            def _(j=j):
                zsrc = z1_ref[...] if j == 1 else zbufs[j % 2][...]
                u_part = ubufs[(j + 1) % 2][rows, :]
                c = s_ref[k - 1 - j]
                res = jnp.dot(
                    pbf_ref[...], zsrc.astype(jnp.bfloat16),
                    preferred_element_type=jnp.float32,
                ) + c * u_part
                if j == k - 1:
                    o_ref[...] = res
                else:
                    zbufs[(j + 1) % 2][rows, :] = res


def _fused_chain(abf, pbf, u1, z1, coefs):
    n = abf.shape[0]
    d = u1.shape[1]
    k = coefs.shape[0] - 1
    npass = 2 * (k - 1)
    nb = n // _BM
    return pl.pallas_call(
        functools.partial(_fused_body, k),
        grid=(npass, nb),
        in_specs=[
            pl.BlockSpec(memory_space=pltpu.SMEM),
            pl.BlockSpec((_BM, n),
                         lambda p, r: (jnp.where(p % 2 == 0, r, nb - 1), 0)),
            pl.BlockSpec((_BM, n),
                         lambda p, r: (jnp.where(p % 2 == 1, r, nb - 1), 0)),
            pl.BlockSpec((n, d), lambda p, r: (0, 0)),
            pl.BlockSpec((n, d), lambda p, r: (0, 0)),
        ],
        out_specs=pl.BlockSpec((_BM, d), lambda p, r: (r, 0)),
        out_shape=jax.ShapeDtypeStruct((n, d), jnp.float32),
        scratch_shapes=[
            pltpu.VMEM((n, d), jnp.float32),
            pltpu.VMEM((n, d), jnp.float32),
            pltpu.VMEM((n, d), jnp.float32),
            pltpu.VMEM((n, d), jnp.float32),
        ],
    )(coefs, abf, pbf, u1, z1)


def kernel(x, adj, poly_item, filter_param):
    k = filter_param.shape[0] - 1
    fp = jax.nn.relu(filter_param)[:, 0]
    combs = jnp.asarray(
        [math.comb(k, i) / (2.0 ** k) for i in range(k + 1)], jnp.float32)
    coefs = fp * combs
    one = jnp.float32(1.0)
    zero = jnp.float32(0.0)

    u1, adj_bf = _apply_cast(adj, x, x, one, zero)
    z1, poly_bf = _apply_cast(poly_item, x, u1, coefs[k], coefs[k - 1])
    return _fused_chain(adj_bf, poly_bf, u1, z1, coefs)


# speedup vs baseline: 2.0553x; 1.1092x over previous
import functools
import math

import jax
import jax.numpy as jnp
from jax.experimental import pallas as pl
from jax.experimental.pallas import tpu as pltpu

_BM = 512


def _cast_pass_body(s_ref, m_ref, v_ref, u_ref, o_ref, mbf_ref):
    a = s_ref[0]
    c = s_ref[1]
    mb = m_ref[...]
    mbf_ref[...] = mb.astype(jnp.bfloat16)
    o_ref[...] = a * jnp.dot(
        mb, v_ref[...], preferred_element_type=jnp.float32
    ) + c * u_ref[...]


def _apply_cast(m, v, u, a, c):
    n = m.shape[0]
    d = v.shape[1]
    s = jnp.stack([a, c]).astype(jnp.float32)
    return pl.pallas_call(
        _cast_pass_body,
        grid=(n // _BM,),
        in_specs=[
            pl.BlockSpec(memory_space=pltpu.SMEM),
            pl.BlockSpec((_BM, n), lambda i: (i, 0)),
            pl.BlockSpec((n, d), lambda i: (0, 0)),
            pl.BlockSpec((_BM, d), lambda i: (i, 0)),
        ],
        out_specs=[
            pl.BlockSpec((_BM, d), lambda i: (i, 0)),
            pl.BlockSpec((_BM, n), lambda i: (i, 0)),
        ],
        out_shape=[
            jax.ShapeDtypeStruct((n, d), jnp.float32),
            jax.ShapeDtypeStruct((n, n), jnp.bfloat16),
        ],
    )(s, m, v, u)


def _fused_body(k, s_ref, abf_ref, pbf_ref, u1_ref, z1_ref, o_ref,
                ub0, ub1, zb0, zb1):
    p = pl.program_id(0)
    r = pl.program_id(1)
    rows = pl.ds(r * _BM, _BM)
    npass = 2 * (k - 1)
    ubufs = (ub0, ub1)
    zbufs = (zb0, zb1)

    for pp in range(npass):
        if pp % 2 == 0:
            i = pp // 2

            @pl.when(p == pp)
            def _(i=i):
                src = u1_ref[...] if i == 0 else ubufs[(i + 1) % 2][...]
                ubufs[i % 2][rows, :] = jnp.dot(
                    abf_ref[...], src.astype(jnp.bfloat16),
                    preferred_element_type=jnp.float32,
                )
        else:
            j = (pp + 1) // 2

            @pl.when(p == pp)
            def _(j=j):
                zsrc = z1_ref[...] if j == 1 else zbufs[j % 2][...]
                u_part = ubufs[(j + 1) % 2][rows, :]
                c = s_ref[k - 1 - j]
                res = jnp.dot(
                    pbf_ref[...], zsrc.astype(jnp.bfloat16),
                    preferred_element_type=jnp.float32,
                ) + c * u_part
                if j == k - 1:
                    o_ref[...] = res
                else:
                    zbufs[(j + 1) % 2][rows, :] = res


def _fused_chain(abf, pbf, u1, z1, coefs):
    n = abf.shape[0]
    d = u1.shape[1]
    k = coefs.shape[0] - 1
    npass = 2 * (k - 1)
    nb = n // _BM
    return pl.pallas_call(
        functools.partial(_fused_body, k),
        grid=(npass, nb),
        in_specs=[
            pl.BlockSpec(memory_space=pltpu.SMEM),
            pl.BlockSpec((_BM, n),
                         lambda p, r: (jnp.where(p % 2 == 0, r, nb - 1), 0)),
            pl.BlockSpec((_BM, n),
                         lambda p, r: (jnp.where(p % 2 == 1, r, nb - 1), 0)),
            pl.BlockSpec((n, d), lambda p, r: (0, 0)),
            pl.BlockSpec((n, d), lambda p, r: (0, 0)),
        ],
        out_specs=pl.BlockSpec((_BM, d), lambda p, r: (r, 0)),
        out_shape=jax.ShapeDtypeStruct((n, d), jnp.float32),
        scratch_shapes=[
            pltpu.VMEM((n, d), jnp.float32),
            pltpu.VMEM((n, d), jnp.float32),
            pltpu.VMEM((n, d), jnp.float32),
            pltpu.VMEM((n, d), jnp.float32),
        ],
    )(coefs, abf, pbf, u1, z1)


def kernel(x, adj, poly_item, filter_param):
    k = filter_param.shape[0] - 1
    fp = jax.nn.relu(filter_param)[:, 0]
    combs = jnp.asarray(
        [math.comb(k, i) / (2.0 ** k) for i in range(k + 1)], jnp.float32)
    coefs = fp * combs
    one = jnp.float32(1.0)
    zero = jnp.float32(0.0)

    u1, adj_bf = _apply_cast(adj, x, x, one, zero)
    z1, poly_bf = _apply_cast(poly_item, x, u1, coefs[k], coefs[k - 1])
    return _fused_chain(adj_bf, poly_bf, u1, z1, coefs)
